# baseline (speedup 1.0000x reference)
import jax
import jax.numpy as jnp
from jax.experimental import pallas as pl
from jax.experimental.pallas import tpu as pltpu


def _lookup_kernel(idx_ref, rot_ref, trans_ref, theta_ref, rho_ref):
    i = idx_ref[0]
    rows = jax.lax.broadcasted_iota(jnp.int32, rot_ref.shape, 0)
    mask = rows == i
    theta_ref[...] = jnp.sum(jnp.where(mask, rot_ref[...], 0.0), axis=0,
                             keepdims=True)
    rho_ref[...] = jnp.sum(jnp.where(mask, trans_ref[...], 0.0), axis=0,
                           keepdims=True)


def kernel(view_ids, rot_weight, trans_weight):
    idx = view_ids[:1].astype(jnp.int32)
    theta, rho = pl.pallas_call(
        _lookup_kernel,
        in_specs=[
            pl.BlockSpec(memory_space=pltpu.SMEM),
            pl.BlockSpec(memory_space=pltpu.VMEM),
            pl.BlockSpec(memory_space=pltpu.VMEM),
        ],
        out_specs=[
            pl.BlockSpec(memory_space=pltpu.VMEM),
            pl.BlockSpec(memory_space=pltpu.VMEM),
        ],
        out_shape=[
            jax.ShapeDtypeStruct((1, 3), jnp.float32),
            jax.ShapeDtypeStruct((1, 3), jnp.float32),
        ],
    )(idx, rot_weight, trans_weight)
    return (theta, rho)


# speedup vs baseline: 1.1251x; 1.1251x over previous
import jax
import jax.numpy as jnp
from jax.experimental import pallas as pl
from jax.experimental.pallas import tpu as pltpu


def _lookup_kernel(idx_ref, rot_ref, trans_ref, theta_ref, rho_ref,
                   sem1, sem2):
    i = idx_ref[0]
    c1 = pltpu.make_async_copy(rot_ref.at[pl.ds(i, 1)], theta_ref, sem1)
    c2 = pltpu.make_async_copy(trans_ref.at[pl.ds(i, 1)], rho_ref, sem2)
    c1.start()
    c2.start()
    c1.wait()
    c2.wait()


def kernel(view_ids, rot_weight, trans_weight):
    idx = view_ids[:1].astype(jnp.int32)
    theta, rho = pl.pallas_call(
        _lookup_kernel,
        in_specs=[
            pl.BlockSpec(memory_space=pltpu.SMEM),
            pl.BlockSpec(memory_space=pl.ANY),
            pl.BlockSpec(memory_space=pl.ANY),
        ],
        out_specs=[
            pl.BlockSpec(memory_space=pl.ANY),
            pl.BlockSpec(memory_space=pl.ANY),
        ],
        out_shape=[
            jax.ShapeDtypeStruct((1, 3), jnp.float32),
            jax.ShapeDtypeStruct((1, 3), jnp.float32),
        ],
        scratch_shapes=[
            pltpu.SemaphoreType.DMA,
            pltpu.SemaphoreType.DMA,
        ],
    )(idx, rot_weight, trans_weight)
    return (theta, rho)
